# baseline (speedup 1.0000x reference)
import functools

import jax
import jax.numpy as jnp
from jax import lax
from jax.experimental import pallas as pl
from jax.experimental.pallas import tpu as pltpu
from jax.experimental.pallas import tpu_sc as plsc

_B = 4096
_L = 20
_H = 64
_V = 1000

_NC = 2
_NS = 16
_NW = _NC * _NS
_BPW = _B // _NW
_G = 16
_NG = _BPW // _G


def _sc_pool_body(ids_hbm, table_hbm, out_hbm, table_v, ids_v, acc_v, out_v):
    wid = lax.axis_index("c") * _NS + lax.axis_index("s")
    base_b = wid * _BPW
    pltpu.sync_copy(table_hbm, table_v)
    pltpu.sync_copy(ids_hbm.at[pl.ds(base_b * _L, _BPW * _L)], ids_v)

    lane = lax.broadcasted_iota(jnp.int32, (_G,), 0)
    lane_l = lane * _L
    lane_h = lane * _H

    def group_body(g, carry):
        goff = g * (_G * _L)

        def ids_at(l):
            return plsc.load_gather(ids_v, [goff + lane_l + l])

        base0 = ids_at(0) * _H
        for c in range(_H):
            acc_v[c] = plsc.load_gather(table_v, [base0 + c])

        def l_body(l, c2):
            basev = ids_at(l) * _H
            for c in range(_H):
                plsc.addupdate(acc_v.at[c], plsc.load_gather(table_v, [basev + c]))
            return c2

        lax.fori_loop(1, _L, l_body, 0)

        zero = lane * 0
        for c in range(_H):
            plsc.store_scatter(out_v, [lane, zero + c], acc_v[c])
        pltpu.sync_copy(out_v, out_hbm.at[pl.ds(base_b + g * _G, _G), :])
        return carry

    lax.fori_loop(0, _NG, group_body, 0)


@functools.partial(
    pl.kernel,
    out_type=jax.ShapeDtypeStruct((_B, _H), jnp.float32),
    mesh=plsc.VectorSubcoreMesh(core_axis_name="c", subcore_axis_name="s"),
    compiler_params=pltpu.CompilerParams(needs_layout_passes=False),
    scratch_types=[
        pltpu.VMEM((_V * _H,), jnp.float32),
        pltpu.VMEM((_BPW * _L,), jnp.int32),
        pltpu.VMEM((_H, _G), jnp.float32),
        pltpu.VMEM((_G, _H), jnp.float32),
    ],
)
def _sc_pool(ids_hbm, table_hbm, out_hbm, table_v, ids_v, acc_v, out_v):
    _sc_pool_body(ids_hbm, table_hbm, out_hbm, table_v, ids_v, acc_v, out_v)


_TB = 256


def _tc_body(x_ref, w_ref, b_ref, out_ref):
    x = x_ref[:, :] * (1.0 / _L)
    y = jnp.dot(x, w_ref[:, :], preferred_element_type=jnp.float32)
    y = y + b_ref[:, :]
    for l in range(_L):
        out_ref[:, l, :] = y


def _tc_project(pooled, W, b2d):
    return pl.pallas_call(
        _tc_body,
        grid=(_B // _TB,),
        in_specs=[
            pl.BlockSpec((_TB, _H), lambda i: (i, 0)),
            pl.BlockSpec((_H, _V), lambda i: (0, 0)),
            pl.BlockSpec((1, _V), lambda i: (0, 0)),
        ],
        out_specs=pl.BlockSpec((_TB, _L, _V), lambda i: (i, 0, 0)),
        out_shape=jax.ShapeDtypeStruct((_B, _L, _V), jnp.float32),
        compiler_params=pltpu.CompilerParams(
            dimension_semantics=("parallel",)),
    )(pooled, W, b2d)


def kernel(input_ids, emb_table, W, b):
    ids_flat = input_ids.astype(jnp.int32).reshape(-1)
    table_flat = emb_table.reshape(-1)
    pooled = _sc_pool(ids_flat, table_flat)
    return _tc_project(pooled, W, b.reshape(1, _V))


# speedup vs baseline: 1.0154x; 1.0154x over previous
import functools

import jax
import jax.numpy as jnp
from jax import lax
from jax.experimental import pallas as pl
from jax.experimental.pallas import tpu as pltpu
from jax.experimental.pallas import tpu_sc as plsc

_B = 4096
_L = 20
_H = 64
_V = 1000

_NC = 2
_NS = 16
_NW = _NC * _NS
_BPW = _B // _NW
_G = 16
_NG = _BPW // _G


def _sc_pool_body(ids_hbm, table_hbm, out_hbm, table_v, ids_v, acc_v, out_v):
    wid = lax.axis_index("c") * _NS + lax.axis_index("s")
    base_b = wid * _BPW
    pltpu.sync_copy(table_hbm, table_v)
    pltpu.sync_copy(ids_hbm.at[pl.ds(base_b * _L, _BPW * _L)], ids_v)

    lane = lax.broadcasted_iota(jnp.int32, (_G,), 0)
    lane_l = lane * _L
    lane_h = lane * _H

    def group_body(g, carry):
        goff = g * (_G * _L)

        def ids_at(l):
            return plsc.load_gather(ids_v, [goff + lane_l + l])

        base0 = ids_at(0) * _H
        for c in range(_H):
            acc_v[c] = plsc.load_gather(table_v, [base0 + c])

        def l_body(l, c2):
            basev = ids_at(l) * _H
            for c in range(_H):
                plsc.addupdate(acc_v.at[c], plsc.load_gather(table_v, [basev + c]))
            return c2

        lax.fori_loop(1, _L, l_body, 0)

        zero = lane * 0
        for c in range(_H):
            plsc.store_scatter(out_v, [lane, zero + c], acc_v[c])
        pltpu.sync_copy(out_v, out_hbm.at[pl.ds(base_b + g * _G, _G), :])
        return carry

    lax.fori_loop(0, _NG, group_body, 0)


@functools.partial(
    pl.kernel,
    out_type=jax.ShapeDtypeStruct((_B, _H), jnp.float32),
    mesh=plsc.VectorSubcoreMesh(core_axis_name="c", subcore_axis_name="s"),
    compiler_params=pltpu.CompilerParams(needs_layout_passes=False),
    scratch_types=[
        pltpu.VMEM((_V * _H,), jnp.float32),
        pltpu.VMEM((_BPW * _L,), jnp.int32),
        pltpu.VMEM((_H, _G), jnp.float32),
        pltpu.VMEM((_G, _H), jnp.float32),
    ],
)
def _sc_pool(ids_hbm, table_hbm, out_hbm, table_v, ids_v, acc_v, out_v):
    _sc_pool_body(ids_hbm, table_hbm, out_hbm, table_v, ids_v, acc_v, out_v)


_TB = 64


def _tc_body(x_ref, w_ref, b_ref, out_ref):
    x = x_ref[:, :] * (1.0 / _L)
    y = jnp.dot(x, w_ref[:, :], preferred_element_type=jnp.float32)
    y = y + b_ref[:, :]
    for l in range(_L):
        out_ref[:, l, :] = y


def _tc_project(pooled, W, b2d):
    return pl.pallas_call(
        _tc_body,
        grid=(_B // _TB,),
        in_specs=[
            pl.BlockSpec((_TB, _H), lambda i: (i, 0)),
            pl.BlockSpec((_H, _V), lambda i: (0, 0)),
            pl.BlockSpec((1, _V), lambda i: (0, 0)),
        ],
        out_specs=pl.BlockSpec((_TB, _L, _V), lambda i: (i, 0, 0)),
        out_shape=jax.ShapeDtypeStruct((_B, _L, _V), jnp.float32),
        compiler_params=pltpu.CompilerParams(
            dimension_semantics=("parallel",)),
    )(pooled, W, b2d)


def kernel(input_ids, emb_table, W, b):
    ids_flat = input_ids.astype(jnp.int32).reshape(-1)
    table_flat = emb_table.reshape(-1)
    pooled = _sc_pool(ids_flat, table_flat)
    return _tc_project(pooled, W, b.reshape(1, _V))
